# baseline (speedup 1.0000x reference)
import functools

import jax
import jax.numpy as jnp
from jax import lax
from jax.experimental import pallas as pl
from jax.experimental.pallas import tpu as pltpu
from jax.experimental.pallas import tpu_sc as plsc

N_ATOMS = 50000
NNEI = 48
M_DIM = 32
AXIS = 4
RCUT = 6.0
RCUT_SMTH = 0.5

NW = 32
TOTAL_IDX = N_ATOMS * NNEI
PER_W = TOTAL_IDX // NW
CHUNK = 5000
N_CHUNKS = PER_W // CHUNK

A_BLK = 400


def _sc_gather(cx, cy, cz, idx):
    mesh = plsc.VectorSubcoreMesh(core_axis_name="c", subcore_axis_name="s")
    out_t = jax.ShapeDtypeStruct((TOTAL_IDX,), jnp.float32)

    @functools.partial(
        pl.kernel,
        mesh=mesh,
        out_type=(out_t, out_t, out_t),
        scratch_types=[
            pltpu.VMEM((CHUNK,), jnp.int32),
            pltpu.VMEM((CHUNK,), jnp.float32),
            pltpu.SemaphoreType.DMA,
        ],
    )
    def k(cx_hbm, cy_hbm, cz_hbm, idx_hbm, ox_hbm, oy_hbm, oz_hbm,
          idx_v, rows_v, sem):
        wid = lax.axis_index("s") * 2 + lax.axis_index("c")
        base = wid * PER_W

        def step(i, carry):
            off = base + i * CHUNK
            pltpu.sync_copy(idx_hbm.at[pl.ds(off, CHUNK)], idx_v)
            for tbl, out in ((cx_hbm, ox_hbm), (cy_hbm, oy_hbm), (cz_hbm, oz_hbm)):
                pltpu.async_copy(tbl.at[idx_v], rows_v, sem).wait()
                pltpu.sync_copy(rows_v, out.at[pl.ds(off, CHUNK)])
            return carry

        lax.fori_loop(0, N_CHUNKS, step, 0)

    return k(cx, cy, cz, idx)


def _switch(r):
    u = (r - RCUT_SMTH) / (RCUT - RCUT_SMTH)
    uu = jnp.clip(u, 0.0, 1.0)
    return uu * uu * uu * (-6.0 * uu * uu + 15.0 * uu - 10.0) + 1.0


def _mm(a, b):
    return jnp.dot(a, b, preferred_element_type=jnp.float32)


def _tc_body(gx_ref, gy_ref, gz_ref, c_ref,
             E0w, b1t0, W2b0, b2t0, C2_0, W3b0, b3t0, E16_0, S16_0, E32_0, S0,
             E1w, b1t1, W2b1, b2t1, C2_1, W3b1, b3t1, E16_1, S16_1, E32_1, S1,
             out_ref):
    c = c_ref[...]
    rx = gx_ref[...] - c[:, 0][:, None]
    ry = gy_ref[...] - c[:, 1][:, None]
    rz = gz_ref[...] - c[:, 2][:, None]
    r2 = rx * rx + ry * ry + rz * rz
    r = jnp.sqrt(jnp.maximum(r2, 1e-12))
    r = jnp.maximum(r, 1e-6)
    inv_r = 1.0 / r
    s = _switch(r) * inv_r
    sr = s * inv_r

    x = jnp.tanh(_mm(s[:, :16], E0w[...]) + b1t0[...][None, :])
    h = jnp.tanh(_mm(x, W2b0[...]) + b2t0[...][None, :])
    x2_0 = _mm(x, C2_0[...]) + h
    h3_0 = jnp.tanh(_mm(x2_0, W3b0[...]) + b3t0[...][None, :])

    x = jnp.tanh(_mm(s[:, 16:], E1w[...]) + b1t1[...][None, :])
    h = jnp.tanh(_mm(x, W2b1[...]) + b2t1[...][None, :])
    x2_1 = _mm(x, C2_1[...]) + h
    h3_1 = jnp.tanh(_mm(x2_1, W3b1[...]) + b3t1[...][None, :])

    inv_nnei = 1.0 / float(NNEI)
    Ts = []
    for Ra in (s, sr * rx, sr * ry, sr * rz):
        R0_16 = _mm(Ra[:, :16], E16_0[...])
        R1_16 = _mm(Ra[:, 16:], E16_1[...])
        Q = _mm(x2_0 * R0_16, S16_0[...]) + _mm(x2_1 * R1_16, S16_1[...])
        R0 = _mm(Ra[:, :16], E32_0[...])
        R1 = _mm(Ra[:, 16:], E32_1[...])
        base = _mm(h3_0 * R0, S0[...]) + _mm(h3_1 * R1, S1[...])
        Ta = (base + jnp.concatenate([Q, Q], axis=1)) * inv_nnei
        Ts.append(Ta)

    Ds = []
    for b in range(4):
        Db = Ts[0] * Ts[0][:, b][:, None]
        for a in range(1, 4):
            Db = Db + Ts[a] * Ts[a][:, b][:, None]
        Ds.append(Db[:, :, None])
    out_ref[...] = jnp.concatenate(Ds, axis=-1)


def _pack_net(W1, b1, W2, b2, W3, b3, sel):
    I = jnp.eye(sel, dtype=jnp.float32)
    Ew = jnp.kron(I, W1)
    b1t = jnp.tile(b1, sel)
    W2b = jnp.kron(I, W2)
    b2t = jnp.tile(b2, sel)
    C2 = jnp.kron(I, jnp.tile(jnp.eye(8, dtype=jnp.float32), (1, 2)))
    W3b = jnp.kron(I, W3)
    b3t = jnp.tile(b3, sel)
    E16 = jnp.kron(I, jnp.ones((1, 16), jnp.float32))
    S16 = jnp.tile(jnp.eye(16, dtype=jnp.float32), (sel, 1))
    E32 = jnp.kron(I, jnp.ones((1, M_DIM), jnp.float32))
    S = jnp.tile(jnp.eye(M_DIM, dtype=jnp.float32), (sel, 1))
    return (Ew, b1t, W2b, b2t, C2, W3b, b3t, E16, S16, E32, S)


def _tc_compute(gx, gy, gz, coord4, packed):
    grid = (N_ATOMS // A_BLK,)
    in_specs = [
        pl.BlockSpec((A_BLK, NNEI), lambda i: (i, 0)),
        pl.BlockSpec((A_BLK, NNEI), lambda i: (i, 0)),
        pl.BlockSpec((A_BLK, NNEI), lambda i: (i, 0)),
        pl.BlockSpec((A_BLK, 4), lambda i: (i, 0)),
    ]
    for p in packed:
        if p.ndim == 2:
            in_specs.append(pl.BlockSpec(p.shape, lambda i: (0, 0)))
        else:
            in_specs.append(pl.BlockSpec(p.shape, lambda i: (0,)))
    return pl.pallas_call(
        _tc_body,
        grid=grid,
        in_specs=in_specs,
        out_specs=pl.BlockSpec((A_BLK, M_DIM, 4), lambda i: (i, 0, 0)),
        out_shape=jax.ShapeDtypeStruct((N_ATOMS, M_DIM, 4), jnp.float32),
    )(gx, gy, gz, coord4, *packed)


def kernel(coord, atype, nlist,
           W1_0, b1_0, W2_0, b2_0, W3_0, b3_0,
           W1_1, b1_1, W2_1, b2_1, W3_1, b3_1):
    del atype
    idx = jnp.asarray(nlist, jnp.int32).reshape(-1)
    gx, gy, gz = _sc_gather(coord[:, 0], coord[:, 1], coord[:, 2], idx)
    coord4 = jnp.pad(coord, ((0, 0), (0, 1)))
    packed = (_pack_net(W1_0, b1_0, W2_0, b2_0, W3_0, b3_0, 16)
              + _pack_net(W1_1, b1_1, W2_1, b2_1, W3_1, b3_1, 32))
    out3 = _tc_compute(gx.reshape(N_ATOMS, NNEI),
                       gy.reshape(N_ATOMS, NNEI),
                       gz.reshape(N_ATOMS, NNEI),
                       coord4, packed)
    return out3.reshape(N_ATOMS, M_DIM * AXIS)


# speedup vs baseline: 1.0118x; 1.0118x over previous
import functools

import jax
import jax.numpy as jnp
from jax import lax
from jax.experimental import pallas as pl
from jax.experimental.pallas import tpu as pltpu
from jax.experimental.pallas import tpu_sc as plsc

N_ATOMS = 50000
NNEI = 48
M_DIM = 32
AXIS = 4
RCUT = 6.0
RCUT_SMTH = 0.5

NW = 32
TOTAL_IDX = N_ATOMS * NNEI
PER_W = TOTAL_IDX // NW
CHUNK = 5000
N_CHUNKS = PER_W // CHUNK

A_BLK = 400


def _sc_gather(cx, cy, cz, idx):
    mesh = plsc.VectorSubcoreMesh(core_axis_name="c", subcore_axis_name="s")
    out_t = jax.ShapeDtypeStruct((TOTAL_IDX,), jnp.float32)

    @functools.partial(
        pl.kernel,
        mesh=mesh,
        out_type=(out_t, out_t, out_t),
        scratch_types=[
            pltpu.VMEM((CHUNK,), jnp.int32),
            pltpu.VMEM((CHUNK,), jnp.float32),
            pltpu.SemaphoreType.DMA,
        ],
    )
    def k(cx_hbm, cy_hbm, cz_hbm, idx_hbm, ox_hbm, oy_hbm, oz_hbm,
          idx_v, rows_v, sem):
        wid = lax.axis_index("s") * 2 + lax.axis_index("c")
        base = wid * PER_W

        def step(i, carry):
            off = base + i * CHUNK
            pltpu.sync_copy(idx_hbm.at[pl.ds(off, CHUNK)], idx_v)
            for tbl, out in ((cx_hbm, ox_hbm), (cy_hbm, oy_hbm), (cz_hbm, oz_hbm)):
                pltpu.async_copy(tbl.at[idx_v], rows_v, sem).wait()
                pltpu.sync_copy(rows_v, out.at[pl.ds(off, CHUNK)])
            return carry

        lax.fori_loop(0, N_CHUNKS, step, 0)

    return k(cx, cy, cz, idx)


def _switch(r):
    u = (r - RCUT_SMTH) / (RCUT - RCUT_SMTH)
    uu = jnp.clip(u, 0.0, 1.0)
    return uu * uu * uu * (-6.0 * uu * uu + 15.0 * uu - 10.0) + 1.0


def _mm(a, b):
    return jnp.dot(a, b, preferred_element_type=jnp.float32)


def _tc_body(gx_ref, gy_ref, gz_ref, c_ref,
             E0w, b1t0, W2b0, b2t0, C2_0, W3b0, b3t0, C3_0, E32_0, S0,
             E1w, b1t1, W2b1, b2t1, C2_1, W3b1, b3t1, C3_1, E32_1, S1,
             out_ref):
    c = c_ref[...]
    rx = gx_ref[...] - c[:, 0][:, None]
    ry = gy_ref[...] - c[:, 1][:, None]
    rz = gz_ref[...] - c[:, 2][:, None]
    r2 = rx * rx + ry * ry + rz * rz
    r = jnp.sqrt(jnp.maximum(r2, 1e-12))
    r = jnp.maximum(r, 1e-6)
    inv_r = 1.0 / r
    s = _switch(r) * inv_r
    sr = s * inv_r

    x = jnp.tanh(_mm(s[:, :16], E0w[...]) + b1t0[...][None, :])
    h = jnp.tanh(_mm(x, W2b0[...]) + b2t0[...][None, :])
    x = _mm(x, C2_0[...]) + h
    h = jnp.tanh(_mm(x, W3b0[...]) + b3t0[...][None, :])
    G0p = _mm(x, C3_0[...]) + h

    x = jnp.tanh(_mm(s[:, 16:], E1w[...]) + b1t1[...][None, :])
    h = jnp.tanh(_mm(x, W2b1[...]) + b2t1[...][None, :])
    x = _mm(x, C2_1[...]) + h
    h = jnp.tanh(_mm(x, W3b1[...]) + b3t1[...][None, :])
    G1p = _mm(x, C3_1[...]) + h

    inv_nnei = 1.0 / float(NNEI)
    Ts = []
    for Ra in (s, sr * rx, sr * ry, sr * rz):
        R0 = _mm(Ra[:, :16], E32_0[...])
        R1 = _mm(Ra[:, 16:], E32_1[...])
        Ta = (_mm(G0p * R0, S0[...]) + _mm(G1p * R1, S1[...])) * inv_nnei
        Ts.append(Ta)

    Ds = []
    for b in range(4):
        Db = Ts[0] * Ts[0][:, b][:, None]
        for a in range(1, 4):
            Db = Db + Ts[a] * Ts[a][:, b][:, None]
        Ds.append(Db[:, :, None])
    out_ref[...] = jnp.concatenate(Ds, axis=-1)


def _pack_net(W1, b1, W2, b2, W3, b3, sel):
    I = jnp.eye(sel, dtype=jnp.float32)
    Ew = jnp.kron(I, W1)
    b1t = jnp.tile(b1, sel)
    W2b = jnp.kron(I, W2)
    b2t = jnp.tile(b2, sel)
    C2 = jnp.kron(I, jnp.tile(jnp.eye(8, dtype=jnp.float32), (1, 2)))
    W3b = jnp.kron(I, W3)
    b3t = jnp.tile(b3, sel)
    C3 = jnp.kron(I, jnp.tile(jnp.eye(16, dtype=jnp.float32), (1, 2)))
    E32 = jnp.kron(I, jnp.ones((1, M_DIM), jnp.float32))
    S = jnp.tile(jnp.eye(M_DIM, dtype=jnp.float32), (sel, 1))
    return (Ew, b1t, W2b, b2t, C2, W3b, b3t, C3, E32, S)


def _tc_compute(gx, gy, gz, coord4, packed):
    grid = (N_ATOMS // A_BLK,)
    in_specs = [
        pl.BlockSpec((A_BLK, NNEI), lambda i: (i, 0)),
        pl.BlockSpec((A_BLK, NNEI), lambda i: (i, 0)),
        pl.BlockSpec((A_BLK, NNEI), lambda i: (i, 0)),
        pl.BlockSpec((A_BLK, 4), lambda i: (i, 0)),
    ]
    for p in packed:
        if p.ndim == 2:
            in_specs.append(pl.BlockSpec(p.shape, lambda i: (0, 0)))
        else:
            in_specs.append(pl.BlockSpec(p.shape, lambda i: (0,)))
    return pl.pallas_call(
        _tc_body,
        grid=grid,
        in_specs=in_specs,
        out_specs=pl.BlockSpec((A_BLK, M_DIM, 4), lambda i: (i, 0, 0)),
        out_shape=jax.ShapeDtypeStruct((N_ATOMS, M_DIM, 4), jnp.float32),
    )(gx, gy, gz, coord4, *packed)


def kernel(coord, atype, nlist,
           W1_0, b1_0, W2_0, b2_0, W3_0, b3_0,
           W1_1, b1_1, W2_1, b2_1, W3_1, b3_1):
    del atype
    idx = jnp.asarray(nlist, jnp.int32).reshape(-1)
    gx, gy, gz = _sc_gather(coord[:, 0], coord[:, 1], coord[:, 2], idx)
    coord4 = jnp.pad(coord, ((0, 0), (0, 1)))
    packed = (_pack_net(W1_0, b1_0, W2_0, b2_0, W3_0, b3_0, 16)
              + _pack_net(W1_1, b1_1, W2_1, b2_1, W3_1, b3_1, 32))
    out3 = _tc_compute(gx.reshape(N_ATOMS, NNEI),
                       gy.reshape(N_ATOMS, NNEI),
                       gz.reshape(N_ATOMS, NNEI),
                       coord4, packed)
    return out3.reshape(N_ATOMS, M_DIM * AXIS)
